# baseline (speedup 1.0000x reference)
import jax
import jax.numpy as jnp
from jax import lax
from jax.experimental import pallas as pl
from jax.experimental.pallas import tpu as pltpu
from jax.experimental.pallas import tpu_sc as plsc

B = 16384
CTX = 20
NOISE = 5
VD = 64
NC = 2
NS = 16
NW = NC * NS
BPW = B // NW
CB = 64
NCHUNK = BPW // CB
NPAIR = CB * NOISE
NGRP = NPAIR // 16


def _sc_body(ctxt_ref, doc_ref, noiset_ref, d_tab, w_tab, ot_tab, out_ref,
             ctxf_buf, noisef_buf, ctxt_idx, noise_idx, doc_idx,
             x_bufs, g_bufs, out_buf, dsems, wsems, gsems):
    wid = lax.axis_index("s") * NC + lax.axis_index("c")
    lane = lax.iota(jnp.int32, 16)

    for c in range(CTX):
        pltpu.sync_copy(ctxt_ref.at[c, pl.ds(wid * BPW, BPW)],
                        ctxf_buf.at[pl.ds(c * BPW, BPW)])
    for n in range(NOISE):
        pltpu.sync_copy(noiset_ref.at[n, pl.ds(wid * BPW, BPW)],
                        noisef_buf.at[pl.ds(n * BPW, BPW)])
    pltpu.sync_copy(doc_ref.at[pl.ds(wid * BPW, BPW)], doc_idx)

    @pl.loop(0, BPW * CTX // 16)
    def _cv1(i):
        ctxt_idx[pl.ds(i * 16, 16)] = (
            ctxf_buf[pl.ds(i * 16, 16)].astype(jnp.int32))

    @pl.loop(0, BPW * NOISE // 16)
    def _cv2(i):
        noise_idx[pl.ds(i * 16, 16)] = (
            noisef_buf[pl.ds(i * 16, 16)].astype(jnp.int32))

    def d_copy(kk, slot):
        return pltpu.make_async_copy(
            d_tab.at[doc_idx.at[pl.ds(kk * CB, CB)]], x_bufs[slot],
            dsems[slot])

    def w_src(kk, c):
        return w_tab.at[ctxt_idx.at[pl.ds(c * BPW + kk * CB, CB)]]

    def g_tr(kk, slot, n):
        return pltpu.make_async_copy(
            ot_tab.at[noise_idx.at[pl.ds(n * BPW + kk * CB, CB)]],
            g_bufs[slot].at[pl.ds(n * CB, CB)], gsems[slot])

    def issue_adds(kk, slot):
        d_copy(kk, slot).wait()
        for c in range(CTX):
            pltpu.async_copy(w_src(kk, c), x_bufs[slot], wsems[slot],
                             add=True)
        for n in range(NOISE):
            g_tr(kk, slot, n).start()

    def drain_adds(kk, slot):
        for c in range(CTX):
            pltpu.make_async_copy(w_src(kk, c), x_bufs[slot],
                                  wsems[slot]).wait()
        for n in range(NOISE):
            g_tr(kk, slot, n).wait()

    def compute(kk, slot):
        x_buf, g_buf = x_bufs[slot], g_bufs[slot]
        perms = [lane ^ jnp.full((16,), m, jnp.int32) for m in (8, 4, 2, 1)]

        @pl.loop(0, NGRP)
        def _pg(g):
            out_vec = jnp.zeros((16,), jnp.float32)
            for j in range(16):
                p = g * 16 + j
                b = lax.div(p, NOISE)
                n = p - b * NOISE
                gr = n * CB + b
                s = x_buf[b, pl.ds(0, 16)] * g_buf[gr, pl.ds(0, 16)]
                for q in range(1, 4):
                    s = s + (x_buf[b, pl.ds(q * 16, 16)]
                             * g_buf[gr, pl.ds(q * 16, 16)])
                for pm in perms:
                    s = s + jax.lax.gather(
                        s, pm[:, None],
                        jax.lax.GatherDimensionNumbers(
                            offset_dims=(), collapsed_slice_dims=(0,),
                            start_index_map=(0,)),
                        (1,),
                        mode=jax.lax.GatherScatterMode.PROMISE_IN_BOUNDS)
                out_vec = jnp.where(lane == jnp.full((16,), j, jnp.int32),
                                    s, out_vec)
            out_buf[pl.ds(g * 16, 16)] = out_vec

        pltpu.sync_copy(out_buf,
                        out_ref.at[pl.ds(wid * BPW * NOISE + kk * NPAIR,
                                         NPAIR)])

    d_copy(0, 0).start()
    issue_adds(0, 0)
    d_copy(1, 1).start()

    @pl.loop(0, NCHUNK, step=2)
    def _pair(k):
        drain_adds(k, 0)
        issue_adds(k + 1, 1)
        compute(k, 0)

        @pl.when(k + 2 < NCHUNK)
        def _():
            d_copy(k + 2, 0).start()

        drain_adds(k + 1, 1)

        @pl.when(k + 2 < NCHUNK)
        def _():
            issue_adds(k + 2, 0)

        compute(k + 1, 1)

        @pl.when(k + 3 < NCHUNK)
        def _():
            d_copy(k + 3, 1).start()


@jax.jit
def _dm_forward(ctxt, doc_ids, noiset, D, W, OT):
    mesh = plsc.VectorSubcoreMesh(core_axis_name="c", subcore_axis_name="s",
                                  num_cores=NC, num_subcores=NS)
    f = pl.kernel(
        _sc_body,
        out_type=jax.ShapeDtypeStruct((B * NOISE,), jnp.float32),
        mesh=mesh,
        scratch_types=[
            pltpu.VMEM((BPW * CTX,), jnp.float32),
            pltpu.VMEM((BPW * NOISE,), jnp.float32),
            pltpu.VMEM((BPW * CTX,), jnp.int32),
            pltpu.VMEM((BPW * NOISE,), jnp.int32),
            pltpu.VMEM((BPW,), jnp.int32),
            [pltpu.VMEM((CB, VD), jnp.float32) for _ in range(2)],
            [pltpu.VMEM((NPAIR, VD), jnp.float32) for _ in range(2)],
            pltpu.VMEM((NPAIR,), jnp.float32),
            [pltpu.SemaphoreType.DMA for _ in range(2)],
            [pltpu.SemaphoreType.DMA for _ in range(2)],
            [pltpu.SemaphoreType.DMA for _ in range(2)],
        ],
        compiler_params=pltpu.CompilerParams(use_tc_tiling_on_sc=False,
                                             needs_layout_passes=False),
    )
    return f(ctxt, doc_ids, noiset, D, W, OT)


def kernel(context_ids, doc_ids, target_noise_ids, D, W, O):
    ctxf = lax.optimization_barrier(context_ids.astype(jnp.float32))
    noisef = lax.optimization_barrier(target_noise_ids.astype(jnp.float32))
    out = _dm_forward(ctxf.T, doc_ids, noisef.T, D, W, O.T)
    return out.reshape(B, NOISE)


# speedup vs baseline: 1.0234x; 1.0234x over previous
import jax
import jax.numpy as jnp
from jax import lax
from jax.experimental import pallas as pl
from jax.experimental.pallas import tpu as pltpu
from jax.experimental.pallas import tpu_sc as plsc

B = 16384
CTX = 20
NOISE = 5
VD = 64
NC = 2
NS = 16
NW = NC * NS
BPW = B // NW
CB = 64
NCHUNK = BPW // CB
NPAIR = CB * NOISE
NGRP = NPAIR // 16


def _sc_body(ctxt_ref, doc_ref, noiset_ref, d_tab, w_tab, ot_tab, out_ref,
             ctxt_idx, noise_idx, doc_idx, x_bufs, g_bufs, out_buf,
             dsems, wsems, gsems):
    wid = lax.axis_index("s") * NC + lax.axis_index("c")
    lane = lax.iota(jnp.int32, 16)

    for c in range(CTX):
        pltpu.sync_copy(ctxt_ref.at[c, pl.ds(wid * BPW, BPW)],
                        ctxt_idx.at[pl.ds(c * BPW, BPW)])
    for n in range(NOISE):
        pltpu.sync_copy(noiset_ref.at[n, pl.ds(wid * BPW, BPW)],
                        noise_idx.at[pl.ds(n * BPW, BPW)])
    pltpu.sync_copy(doc_ref.at[pl.ds(wid * BPW, BPW)], doc_idx)

    def d_copy(kk, slot):
        return pltpu.make_async_copy(
            d_tab.at[doc_idx.at[pl.ds(kk * CB, CB)]], x_bufs[slot],
            dsems[slot])

    def w_src(kk, c):
        return w_tab.at[ctxt_idx.at[pl.ds(c * BPW + kk * CB, CB)]]

    def g_tr(kk, slot, n):
        return pltpu.make_async_copy(
            ot_tab.at[noise_idx.at[pl.ds(n * BPW + kk * CB, CB)]],
            g_bufs[slot].at[pl.ds(n * CB, CB)], gsems[slot])

    def issue_adds(kk, slot):
        d_copy(kk, slot).wait()
        for c in range(CTX):
            pltpu.async_copy(w_src(kk, c), x_bufs[slot], wsems[slot],
                             add=True)
        for n in range(NOISE):
            g_tr(kk, slot, n).start()

    def drain_adds(kk, slot):
        for c in range(CTX):
            pltpu.make_async_copy(w_src(kk, c), x_bufs[slot],
                                  wsems[slot]).wait()
        for n in range(NOISE):
            g_tr(kk, slot, n).wait()

    def compute(kk, slot):
        x_buf, g_buf = x_bufs[slot], g_bufs[slot]
        perms = [lane ^ jnp.full((16,), m, jnp.int32) for m in (8, 4, 2, 1)]

        @pl.loop(0, NGRP)
        def _pg(g):
            out_vec = jnp.zeros((16,), jnp.float32)
            for j in range(16):
                p = g * 16 + j
                b = lax.div(p, NOISE)
                n = p - b * NOISE
                gr = n * CB + b
                s = x_buf[b, pl.ds(0, 16)] * g_buf[gr, pl.ds(0, 16)]
                for q in range(1, 4):
                    s = s + (x_buf[b, pl.ds(q * 16, 16)]
                             * g_buf[gr, pl.ds(q * 16, 16)])
                for pm in perms:
                    s = s + jax.lax.gather(
                        s, pm[:, None],
                        jax.lax.GatherDimensionNumbers(
                            offset_dims=(), collapsed_slice_dims=(0,),
                            start_index_map=(0,)),
                        (1,),
                        mode=jax.lax.GatherScatterMode.PROMISE_IN_BOUNDS)
                out_vec = jnp.where(lane == jnp.full((16,), j, jnp.int32),
                                    s, out_vec)
            out_buf[pl.ds(g * 16, 16)] = out_vec

        pltpu.sync_copy(out_buf,
                        out_ref.at[pl.ds(wid * BPW * NOISE + kk * NPAIR,
                                         NPAIR)])

    d_copy(0, 0).start()
    issue_adds(0, 0)
    d_copy(1, 1).start()

    @pl.loop(0, NCHUNK, step=2)
    def _pair(k):
        drain_adds(k, 0)
        issue_adds(k + 1, 1)
        compute(k, 0)

        @pl.when(k + 2 < NCHUNK)
        def _():
            d_copy(k + 2, 0).start()

        drain_adds(k + 1, 1)

        @pl.when(k + 2 < NCHUNK)
        def _():
            issue_adds(k + 2, 0)

        compute(k + 1, 1)

        @pl.when(k + 3 < NCHUNK)
        def _():
            d_copy(k + 3, 1).start()


@jax.jit
def _dm_forward(ctxt, doc_ids, noiset, D, W, OT):
    mesh = plsc.VectorSubcoreMesh(core_axis_name="c", subcore_axis_name="s",
                                  num_cores=NC, num_subcores=NS)
    f = pl.kernel(
        _sc_body,
        out_type=jax.ShapeDtypeStruct((B * NOISE,), jnp.float32),
        mesh=mesh,
        scratch_types=[
            pltpu.VMEM((BPW * CTX,), jnp.int32),
            pltpu.VMEM((BPW * NOISE,), jnp.int32),
            pltpu.VMEM((BPW,), jnp.int32),
            [pltpu.VMEM((CB, VD), jnp.float32) for _ in range(2)],
            [pltpu.VMEM((NPAIR, VD), jnp.float32) for _ in range(2)],
            pltpu.VMEM((NPAIR,), jnp.float32),
            [pltpu.SemaphoreType.DMA for _ in range(2)],
            [pltpu.SemaphoreType.DMA for _ in range(2)],
            [pltpu.SemaphoreType.DMA for _ in range(2)],
        ],
        compiler_params=pltpu.CompilerParams(use_tc_tiling_on_sc=False,
                                             needs_layout_passes=False),
    )
    return f(ctxt, doc_ids, noiset, D, W, OT)


def kernel(context_ids, doc_ids, target_noise_ids, D, W, O):
    out = _dm_forward(context_ids.T, doc_ids, target_noise_ids.T,
                      D, W, O.T)
    return out.reshape(B, NOISE)
